# initial kernel scaffold (unmeasured)
import jax
import jax.numpy as jnp
from jax import lax
from jax.experimental import pallas as pl
from jax.experimental.pallas import tpu as pltpu

N_DEV = 16


def kernel(x, W):
    t, d = x.shape
    _, v_per = W.shape
    v_total = N_DEV * v_per

    def body(x_ref, w_ref, out_ref, send_sems, recv_sems):
        my = lax.axis_index("i")
        left = lax.rem(my + N_DEV - 1, N_DEV)
        right = lax.rem(my + 1, N_DEV)

        logits = jnp.dot(
            x_ref[:, :], w_ref[:, :], preferred_element_type=jnp.float32
        )
        out_ref[:, pl.ds(my * v_per, v_per)] = logits

        barrier_sem = pltpu.get_barrier_semaphore()
        for nbr in (left, right):
            pl.semaphore_signal(
                barrier_sem, inc=1,
                device_id=(nbr,), device_id_type=pl.DeviceIdType.MESH,
            )
        pl.semaphore_wait(barrier_sem, 2)

        for h in range(N_DEV - 1):
            origin = lax.rem(my - h + 2 * N_DEV, N_DEV)
            sl = pl.ds(origin * v_per, v_per)
            rdma = pltpu.make_async_remote_copy(
                src_ref=out_ref.at[:, sl],
                dst_ref=out_ref.at[:, sl],
                send_sem=send_sems.at[h],
                recv_sem=recv_sems.at[h],
                device_id=(right,),
                device_id_type=pl.DeviceIdType.MESH,
            )
            rdma.start()
            rdma.wait()

        m = jnp.full((t, 1), -jnp.inf, dtype=jnp.float32)
        for c in range(N_DEV):
            chunk = out_ref[:, c * v_per:(c + 1) * v_per]
            m = jnp.maximum(m, jnp.max(chunk, axis=-1, keepdims=True))
        s = jnp.zeros((t, 1), dtype=jnp.float32)
        for c in range(N_DEV):
            e = jnp.exp(out_ref[:, c * v_per:(c + 1) * v_per] - m)
            out_ref[:, c * v_per:(c + 1) * v_per] = e
            s = s + jnp.sum(e, axis=-1, keepdims=True)
        inv = 1.0 / s
        for c in range(N_DEV):
            out_ref[:, c * v_per:(c + 1) * v_per] = (
                out_ref[:, c * v_per:(c + 1) * v_per] * inv
            )

    return pl.pallas_call(
        body,
        out_shape=jax.ShapeDtypeStruct((t, v_total), jnp.float32),
        in_specs=[
            pl.BlockSpec(memory_space=pltpu.VMEM),
            pl.BlockSpec(memory_space=pltpu.VMEM),
        ],
        out_specs=pl.BlockSpec(memory_space=pltpu.VMEM),
        scratch_shapes=[
            pltpu.SemaphoreType.DMA((N_DEV - 1,)),
            pltpu.SemaphoreType.DMA((N_DEV - 1,)),
        ],
        compiler_params=pltpu.CompilerParams(collective_id=0),
    )(x, W)


# baseline (device time: 918692 ns/iter reference)
import jax
import jax.numpy as jnp
from jax import lax
from jax.experimental import pallas as pl
from jax.experimental.pallas import tpu as pltpu

N_DEV = 16


def kernel(x, W):
    t, d = x.shape
    _, v_per = W.shape
    v_total = N_DEV * v_per

    def body(x_ref, w_ref, out_ref, cbuf, send_sems, recv_sems, csem):
        my = lax.axis_index("i")
        left = lax.rem(my + N_DEV - 1, N_DEV)
        right = lax.rem(my + 1, N_DEV)

        cbuf[0] = jnp.dot(
            x_ref[:, :], w_ref[:, :], preferred_element_type=jnp.float32
        )
        cp = pltpu.make_async_copy(
            cbuf.at[0], out_ref.at[:, pl.ds(my * v_per, v_per)], csem
        )
        cp.start()
        cp.wait()

        barrier_sem = pltpu.get_barrier_semaphore()
        for nbr in (left, right):
            pl.semaphore_signal(
                barrier_sem, inc=1,
                device_id=(nbr,), device_id_type=pl.DeviceIdType.MESH,
            )
        pl.semaphore_wait(barrier_sem, 2)

        for h in range(N_DEV - 1):
            origin = lax.rem(my - h + 2 * N_DEV, N_DEV)
            sl = pl.ds(origin * v_per, v_per)
            rdma = pltpu.make_async_remote_copy(
                src_ref=out_ref.at[:, sl],
                dst_ref=out_ref.at[:, sl],
                send_sem=send_sems.at[h],
                recv_sem=recv_sems.at[h],
                device_id=(right,),
                device_id_type=pl.DeviceIdType.MESH,
            )
            rdma.start()
            rdma.wait()

        def load(c):
            cp = pltpu.make_async_copy(
                out_ref.at[:, pl.ds(c * v_per, v_per)], cbuf.at[0], csem
            )
            cp.start()
            cp.wait()
            return cbuf[0]

        m = jnp.full((t, 1), -jnp.inf, dtype=jnp.float32)
        for c in range(N_DEV):
            m = jnp.maximum(
                m, jnp.max(load(c), axis=-1, keepdims=True)
            )
        s = jnp.zeros((t, 1), dtype=jnp.float32)
        for c in range(N_DEV):
            s = s + jnp.sum(
                jnp.exp(load(c) - m), axis=-1, keepdims=True
            )
        inv = 1.0 / s
        for c in range(N_DEV):
            cbuf[1] = jnp.exp(load(c) - m) * inv
            cp = pltpu.make_async_copy(
                cbuf.at[1], out_ref.at[:, pl.ds(c * v_per, v_per)], csem
            )
            cp.start()
            cp.wait()

    return pl.pallas_call(
        body,
        out_shape=jax.ShapeDtypeStruct((t, v_total), jnp.float32),
        in_specs=[
            pl.BlockSpec(memory_space=pltpu.VMEM),
            pl.BlockSpec(memory_space=pltpu.VMEM),
        ],
        out_specs=pl.BlockSpec(memory_space=pltpu.MemorySpace.HBM),
        scratch_shapes=[
            pltpu.VMEM((2, t, v_per), jnp.float32),
            pltpu.SemaphoreType.DMA((N_DEV - 1,)),
            pltpu.SemaphoreType.DMA((N_DEV - 1,)),
            pltpu.SemaphoreType.DMA,
        ],
        compiler_params=pltpu.CompilerParams(collective_id=0),
    )(x, W)


# device time: 793711 ns/iter; 1.1575x vs baseline; 1.1575x over previous
import jax
import jax.numpy as jnp
from jax import lax
from jax.experimental import pallas as pl
from jax.experimental.pallas import tpu as pltpu

N_DEV = 16


def kernel(x, W):
    t, d = x.shape
    _, v_per = W.shape
    v_total = N_DEV * v_per

    def body(x_ref, w_ref, out_ref, cbuf, msbuf,
             ms_send, ms_recv, ring_send, ring_recv, csem):
        my = lax.axis_index("i")
        right = lax.rem(my + 1, N_DEV)

        logits = jnp.dot(
            x_ref[:, :], w_ref[:, :], preferred_element_type=jnp.float32
        )

        m_loc = jnp.max(logits, axis=-1, keepdims=True)
        s_loc = jnp.sum(jnp.exp(logits - m_loc), axis=-1, keepdims=True)
        pad = jnp.zeros((t, 126), dtype=jnp.float32)
        msbuf[my] = jnp.concatenate([m_loc, s_loc, pad], axis=-1)

        sends = []
        for k in range(1, N_DEV):
            tgt = lax.rem(my + k, N_DEV)
            r = pltpu.make_async_remote_copy(
                src_ref=msbuf.at[my],
                dst_ref=msbuf.at[my],
                send_sem=ms_send.at[k - 1],
                recv_sem=ms_recv.at[my],
                device_id=(tgt,),
                device_id_type=pl.DeviceIdType.MESH,
            )
            r.start()
            sends.append(r)
        for k in range(1, N_DEV):
            src_dev = lax.rem(my - k + N_DEV, N_DEV)
            recv = pltpu.make_async_remote_copy(
                src_ref=msbuf.at[src_dev],
                dst_ref=msbuf.at[src_dev],
                send_sem=ms_send.at[k - 1],
                recv_sem=ms_recv.at[src_dev],
                device_id=(my,),
                device_id_type=pl.DeviceIdType.MESH,
            )
            recv.wait_recv()
        for r in sends:
            r.wait_send()

        M = msbuf[0, :, 0:1]
        for c in range(1, N_DEV):
            M = jnp.maximum(M, msbuf[c, :, 0:1])
        S = jnp.zeros((t, 1), dtype=jnp.float32)
        for c in range(N_DEV):
            S = S + msbuf[c, :, 1:2] * jnp.exp(msbuf[c, :, 0:1] - M)
        cbuf[0] = jnp.exp(logits - M) * (1.0 / S)
        cp = pltpu.make_async_copy(
            cbuf.at[0], out_ref.at[:, pl.ds(my * v_per, v_per)], csem
        )
        cp.start()
        cp.wait()

        for h in range(N_DEV - 1):
            origin = lax.rem(my - h + 2 * N_DEV, N_DEV)
            sl = pl.ds(origin * v_per, v_per)
            rdma = pltpu.make_async_remote_copy(
                src_ref=out_ref.at[:, sl],
                dst_ref=out_ref.at[:, sl],
                send_sem=ring_send.at[h],
                recv_sem=ring_recv.at[h],
                device_id=(right,),
                device_id_type=pl.DeviceIdType.MESH,
            )
            rdma.start()
            rdma.wait()

    return pl.pallas_call(
        body,
        out_shape=jax.ShapeDtypeStruct((t, v_total), jnp.float32),
        in_specs=[
            pl.BlockSpec(memory_space=pltpu.VMEM),
            pl.BlockSpec(memory_space=pltpu.VMEM),
        ],
        out_specs=pl.BlockSpec(memory_space=pltpu.MemorySpace.HBM),
        scratch_shapes=[
            pltpu.VMEM((1, t, v_per), jnp.float32),
            pltpu.VMEM((N_DEV, t, 128), jnp.float32),
            pltpu.SemaphoreType.DMA((N_DEV - 1,)),
            pltpu.SemaphoreType.DMA((N_DEV,)),
            pltpu.SemaphoreType.DMA((N_DEV - 1,)),
            pltpu.SemaphoreType.DMA((N_DEV - 1,)),
            pltpu.SemaphoreType.DMA,
        ],
    )(x, W)


# device time: 486882 ns/iter; 1.8869x vs baseline; 1.6302x over previous
import jax
import jax.numpy as jnp
from jax import lax
from jax.experimental import pallas as pl
from jax.experimental.pallas import tpu as pltpu

N_DEV = 16


def kernel(x, W):
    t, d = x.shape
    _, v_per = W.shape
    v_total = N_DEV * v_per

    def body(x_ref, w_ref, out_ref, cbuf, msbuf,
             ms_send, ms_recv, ring_send, ring_recv,
             ring2_send, ring2_recv, csem):
        my = lax.axis_index("i")
        right = lax.rem(my + 1, N_DEV)
        left = lax.rem(my + N_DEV - 1, N_DEV)

        logits = jnp.dot(
            x_ref[:, :], w_ref[:, :], preferred_element_type=jnp.float32
        )

        m_loc = jnp.max(logits, axis=-1, keepdims=True)
        s_loc = jnp.sum(jnp.exp(logits - m_loc), axis=-1, keepdims=True)
        pad = jnp.zeros((t, 126), dtype=jnp.float32)
        msbuf[my] = jnp.concatenate([m_loc, s_loc, pad], axis=-1)

        sends = []
        for k in range(1, N_DEV):
            tgt = lax.rem(my + k, N_DEV)
            r = pltpu.make_async_remote_copy(
                src_ref=msbuf.at[my],
                dst_ref=msbuf.at[my],
                send_sem=ms_send.at[k - 1],
                recv_sem=ms_recv.at[my],
                device_id=(tgt,),
                device_id_type=pl.DeviceIdType.MESH,
            )
            r.start()
            sends.append(r)
        for k in range(1, N_DEV):
            src_dev = lax.rem(my - k + N_DEV, N_DEV)
            recv = pltpu.make_async_remote_copy(
                src_ref=msbuf.at[src_dev],
                dst_ref=msbuf.at[src_dev],
                send_sem=ms_send.at[k - 1],
                recv_sem=ms_recv.at[src_dev],
                device_id=(my,),
                device_id_type=pl.DeviceIdType.MESH,
            )
            recv.wait_recv()
        for r in sends:
            r.wait_send()

        M = msbuf[0, :, 0:1]
        for c in range(1, N_DEV):
            M = jnp.maximum(M, msbuf[c, :, 0:1])
        S = jnp.zeros((t, 1), dtype=jnp.float32)
        for c in range(N_DEV):
            S = S + msbuf[c, :, 1:2] * jnp.exp(msbuf[c, :, 0:1] - M)
        cbuf[0] = jnp.exp(logits - M) * (1.0 / S)
        cp = pltpu.make_async_copy(
            cbuf.at[0], out_ref.at[:, pl.ds(my * v_per, v_per)], csem
        )
        cp.start()
        cp.wait()

        v_half = v_per // 2
        for h in range(N_DEV - 1):
            origin_cw = lax.rem(my - h + 2 * N_DEV, N_DEV)
            origin_ccw = lax.rem(my + h, N_DEV)
            sl_cw = pl.ds(origin_cw * v_per, v_half)
            sl_ccw = pl.ds(origin_ccw * v_per + v_half, v_half)
            rdma_cw = pltpu.make_async_remote_copy(
                src_ref=out_ref.at[:, sl_cw],
                dst_ref=out_ref.at[:, sl_cw],
                send_sem=ring_send.at[h],
                recv_sem=ring_recv.at[h],
                device_id=(right,),
                device_id_type=pl.DeviceIdType.MESH,
            )
            rdma_ccw = pltpu.make_async_remote_copy(
                src_ref=out_ref.at[:, sl_ccw],
                dst_ref=out_ref.at[:, sl_ccw],
                send_sem=ring2_send.at[h],
                recv_sem=ring2_recv.at[h],
                device_id=(left,),
                device_id_type=pl.DeviceIdType.MESH,
            )
            rdma_cw.start()
            rdma_ccw.start()
            rdma_cw.wait()
            rdma_ccw.wait()

    return pl.pallas_call(
        body,
        out_shape=jax.ShapeDtypeStruct((t, v_total), jnp.float32),
        in_specs=[
            pl.BlockSpec(memory_space=pltpu.VMEM),
            pl.BlockSpec(memory_space=pltpu.VMEM),
        ],
        out_specs=pl.BlockSpec(memory_space=pltpu.MemorySpace.HBM),
        scratch_shapes=[
            pltpu.VMEM((1, t, v_per), jnp.float32),
            pltpu.VMEM((N_DEV, t, 128), jnp.float32),
            pltpu.SemaphoreType.DMA((N_DEV - 1,)),
            pltpu.SemaphoreType.DMA((N_DEV,)),
            pltpu.SemaphoreType.DMA((N_DEV - 1,)),
            pltpu.SemaphoreType.DMA((N_DEV - 1,)),
            pltpu.SemaphoreType.DMA((N_DEV - 1,)),
            pltpu.SemaphoreType.DMA((N_DEV - 1,)),
            pltpu.SemaphoreType.DMA,
        ],
    )(x, W)


# device time: 486524 ns/iter; 1.8883x vs baseline; 1.0007x over previous
import jax
import jax.numpy as jnp
from jax import lax
from jax.experimental import pallas as pl
from jax.experimental.pallas import tpu as pltpu

N_DEV = 16


def kernel(x, W):
    t, d = x.shape
    _, v_per = W.shape
    v_total = N_DEV * v_per

    def body(x_ref, w_ref, out_ref, cbuf, msbuf,
             ms_send, ms_recv, ring_send, ring_recv,
             ring2_send, ring2_recv, csem):
        my = lax.axis_index("i")
        right = lax.rem(my + 1, N_DEV)
        left = lax.rem(my + N_DEV - 1, N_DEV)

        logits = jnp.dot(
            x_ref[:, :], w_ref[:, :], preferred_element_type=jnp.float32
        )

        m_loc = jnp.max(logits, axis=-1, keepdims=True)
        s_loc = jnp.sum(jnp.exp(logits - m_loc), axis=-1, keepdims=True)
        pad = jnp.zeros((t, 126), dtype=jnp.float32)
        msbuf[my] = jnp.concatenate([m_loc, s_loc, pad], axis=-1)

        sends = []
        for k in range(1, N_DEV):
            tgt = lax.rem(my + k, N_DEV)
            r = pltpu.make_async_remote_copy(
                src_ref=msbuf.at[my],
                dst_ref=msbuf.at[my],
                send_sem=ms_send.at[k - 1],
                recv_sem=ms_recv.at[my],
                device_id=(tgt,),
                device_id_type=pl.DeviceIdType.MESH,
            )
            r.start()
            sends.append(r)
        for k in range(1, N_DEV):
            src_dev = lax.rem(my - k + N_DEV, N_DEV)
            recv = pltpu.make_async_remote_copy(
                src_ref=msbuf.at[src_dev],
                dst_ref=msbuf.at[src_dev],
                send_sem=ms_send.at[k - 1],
                recv_sem=ms_recv.at[src_dev],
                device_id=(my,),
                device_id_type=pl.DeviceIdType.MESH,
            )
            recv.wait_recv()
        for r in sends:
            r.wait_send()

        M = msbuf[0, :, 0:1]
        for c in range(1, N_DEV):
            M = jnp.maximum(M, msbuf[c, :, 0:1])
        S = jnp.zeros((t, 1), dtype=jnp.float32)
        for c in range(N_DEV):
            S = S + msbuf[c, :, 1:2] * jnp.exp(msbuf[c, :, 0:1] - M)
        cbuf[0] = jnp.exp(logits - M) * (1.0 / S)
        cp = pltpu.make_async_copy(
            cbuf.at[0], out_ref.at[:, pl.ds(my * v_per, v_per)], csem
        )
        cp.start()
        cp.wait()

        v_half = v_per // 2

        def mk_hop(h):
            origin_cw = lax.rem(my - h + 2 * N_DEV, N_DEV)
            origin_ccw = lax.rem(my + h, N_DEV)
            sl_cw = pl.ds(origin_cw * v_per, v_half)
            sl_ccw = pl.ds(origin_ccw * v_per + v_half, v_half)
            rdma_cw = pltpu.make_async_remote_copy(
                src_ref=out_ref.at[:, sl_cw],
                dst_ref=out_ref.at[:, sl_cw],
                send_sem=ring_send.at[h],
                recv_sem=ring_recv.at[h],
                device_id=(right,),
                device_id_type=pl.DeviceIdType.MESH,
            )
            rdma_ccw = pltpu.make_async_remote_copy(
                src_ref=out_ref.at[:, sl_ccw],
                dst_ref=out_ref.at[:, sl_ccw],
                send_sem=ring2_send.at[h],
                recv_sem=ring2_recv.at[h],
                device_id=(left,),
                device_id_type=pl.DeviceIdType.MESH,
            )
            return rdma_cw, rdma_ccw

        cws, ccws = [], []
        a, b = mk_hop(0)
        a.start()
        b.start()
        cws.append(a)
        ccws.append(b)
        for h in range(N_DEV - 1):
            cws[h].wait_recv()
            ccws[h].wait_recv()
            if h < N_DEV - 2:
                a, b = mk_hop(h + 1)
                a.start()
                b.start()
                cws.append(a)
                ccws.append(b)
        for h in range(N_DEV - 1):
            cws[h].wait_send()
            ccws[h].wait_send()

    return pl.pallas_call(
        body,
        out_shape=jax.ShapeDtypeStruct((t, v_total), jnp.float32),
        in_specs=[
            pl.BlockSpec(memory_space=pltpu.VMEM),
            pl.BlockSpec(memory_space=pltpu.VMEM),
        ],
        out_specs=pl.BlockSpec(memory_space=pltpu.MemorySpace.HBM),
        scratch_shapes=[
            pltpu.VMEM((1, t, v_per), jnp.float32),
            pltpu.VMEM((N_DEV, t, 128), jnp.float32),
            pltpu.SemaphoreType.DMA((N_DEV - 1,)),
            pltpu.SemaphoreType.DMA((N_DEV,)),
            pltpu.SemaphoreType.DMA((N_DEV - 1,)),
            pltpu.SemaphoreType.DMA((N_DEV - 1,)),
            pltpu.SemaphoreType.DMA((N_DEV - 1,)),
            pltpu.SemaphoreType.DMA((N_DEV - 1,)),
            pltpu.SemaphoreType.DMA,
        ],
    )(x, W)


# device time: 451334 ns/iter; 2.0355x vs baseline; 1.0780x over previous
import jax
import jax.numpy as jnp
from jax import lax
from jax.experimental import pallas as pl
from jax.experimental.pallas import tpu as pltpu

N_DEV = 16


def kernel(x, W):
    t, d = x.shape
    _, v_per = W.shape
    v_total = N_DEV * v_per

    def body(x_ref, w_ref, out_ref, cbuf, msbuf,
             ms_send, ms_recv, ring_send, ring_recv,
             ring2_send, ring2_recv, csem):
        my = lax.axis_index("i")
        right = lax.rem(my + 1, N_DEV)
        left = lax.rem(my + N_DEV - 1, N_DEV)

        logits = jnp.dot(
            x_ref[:, :], w_ref[:, :], preferred_element_type=jnp.float32
        )

        m_loc = jnp.max(logits, axis=-1, keepdims=True)
        s_loc = jnp.sum(jnp.exp(logits - m_loc), axis=-1, keepdims=True)
        pad = jnp.zeros((t, 126), dtype=jnp.float32)
        msbuf[my] = jnp.concatenate([m_loc, s_loc, pad], axis=-1)

        sends = []
        for k in range(1, N_DEV):
            tgt = lax.rem(my + k, N_DEV)
            r = pltpu.make_async_remote_copy(
                src_ref=msbuf.at[my],
                dst_ref=msbuf.at[my],
                send_sem=ms_send.at[k - 1],
                recv_sem=ms_recv.at[my],
                device_id=(tgt,),
                device_id_type=pl.DeviceIdType.MESH,
            )
            r.start()
            sends.append(r)
        for k in range(1, N_DEV):
            src_dev = lax.rem(my - k + N_DEV, N_DEV)
            recv = pltpu.make_async_remote_copy(
                src_ref=msbuf.at[src_dev],
                dst_ref=msbuf.at[src_dev],
                send_sem=ms_send.at[k - 1],
                recv_sem=ms_recv.at[src_dev],
                device_id=(my,),
                device_id_type=pl.DeviceIdType.MESH,
            )
            recv.wait_recv()
        for r in sends:
            r.wait_send()

        M = msbuf[0, :, 0:1]
        for c in range(1, N_DEV):
            M = jnp.maximum(M, msbuf[c, :, 0:1])
        S = jnp.zeros((t, 1), dtype=jnp.float32)
        for c in range(N_DEV):
            S = S + msbuf[c, :, 1:2] * jnp.exp(msbuf[c, :, 0:1] - M)
        cbuf[0] = jnp.exp(logits - M) * (1.0 / S)
        cp = pltpu.make_async_copy(
            cbuf.at[0], out_ref.at[:, pl.ds(my * v_per, v_per)], csem
        )
        cp.start()

        v_half = v_per // 2
        v_q = v_per // 4

        def mk_hop(h, direction, q):
            if direction == 0:
                origin = lax.rem(my - h + 2 * N_DEV, N_DEV)
                col = q * v_q
                tgt = right
                ssem, rsem = ring_send, ring_recv
            else:
                origin = lax.rem(my + h, N_DEV)
                col = v_half + q * v_q
                tgt = left
                ssem, rsem = ring2_send, ring2_recv
            sl = pl.ds(origin * v_per + col, v_q)
            if h == 0:
                src = cbuf.at[0, :, pl.ds(col, v_q)]
            else:
                src = out_ref.at[:, sl]
            return pltpu.make_async_remote_copy(
                src_ref=src,
                dst_ref=out_ref.at[:, sl],
                send_sem=ssem.at[h, q],
                recv_sem=rsem.at[h, q],
                device_id=(tgt,),
                device_id_type=pl.DeviceIdType.MESH,
            )

        streams = {}
        for q in range(2):
            for direction in range(2):
                r = mk_hop(0, direction, q)
                r.start()
                streams[(direction, q)] = [r]
        for h in range(N_DEV - 1):
            for q in range(2):
                for direction in range(2):
                    streams[(direction, q)][h].wait_recv()
                    if h < N_DEV - 2:
                        r = mk_hop(h + 1, direction, q)
                        r.start()
                        streams[(direction, q)].append(r)
        for lst in streams.values():
            for r in lst:
                r.wait_send()
        cp.wait()

    return pl.pallas_call(
        body,
        out_shape=jax.ShapeDtypeStruct((t, v_total), jnp.float32),
        in_specs=[
            pl.BlockSpec(memory_space=pltpu.VMEM),
            pl.BlockSpec(memory_space=pltpu.VMEM),
        ],
        out_specs=pl.BlockSpec(memory_space=pltpu.MemorySpace.HBM),
        scratch_shapes=[
            pltpu.VMEM((1, t, v_per), jnp.float32),
            pltpu.VMEM((N_DEV, t, 128), jnp.float32),
            pltpu.SemaphoreType.DMA((N_DEV - 1,)),
            pltpu.SemaphoreType.DMA((N_DEV,)),
            pltpu.SemaphoreType.DMA((N_DEV - 1, 2)),
            pltpu.SemaphoreType.DMA((N_DEV - 1, 2)),
            pltpu.SemaphoreType.DMA((N_DEV - 1, 2)),
            pltpu.SemaphoreType.DMA((N_DEV - 1, 2)),
            pltpu.SemaphoreType.DMA,
        ],
    )(x, W)


# device time: 446745 ns/iter; 2.0564x vs baseline; 1.0103x over previous
import jax
import jax.numpy as jnp
from jax import lax
from jax.experimental import pallas as pl
from jax.experimental.pallas import tpu as pltpu

N_DEV = 16


def kernel(x, W):
    t, d = x.shape
    _, v_per = W.shape
    v_total = N_DEV * v_per

    def body(x_ref, w_ref, out_ref, cbuf, msbuf,
             ms_send, ms_recv, ring_send, ring_recv,
             ring2_send, ring2_recv, csem):
        my = lax.axis_index("i")
        right = lax.rem(my + 1, N_DEV)
        left = lax.rem(my + N_DEV - 1, N_DEV)

        barrier_sem = pltpu.get_barrier_semaphore()
        for k in range(1, N_DEV):
            pl.semaphore_signal(
                barrier_sem, inc=1,
                device_id=(lax.rem(my + k, N_DEV),),
                device_id_type=pl.DeviceIdType.MESH,
            )

        logits = jnp.dot(
            x_ref[:, :], w_ref[:, :], preferred_element_type=jnp.float32
        )
        m_loc = jnp.max(logits, axis=-1, keepdims=True)
        e_loc = jnp.exp(logits - m_loc)
        s_loc = jnp.sum(e_loc, axis=-1, keepdims=True)
        cbuf[0] = e_loc

        pad = jnp.zeros((t, 126), dtype=jnp.float32)
        msbuf[my] = jnp.concatenate([m_loc, s_loc, pad], axis=-1)

        pl.semaphore_wait(barrier_sem, N_DEV - 1)

        sends = []
        for k in range(1, N_DEV):
            tgt = lax.rem(my + k, N_DEV)
            r = pltpu.make_async_remote_copy(
                src_ref=msbuf.at[my],
                dst_ref=msbuf.at[my],
                send_sem=ms_send.at[k - 1],
                recv_sem=ms_recv.at[my],
                device_id=(tgt,),
                device_id_type=pl.DeviceIdType.MESH,
            )
            r.start()
            sends.append(r)
        for k in range(1, N_DEV):
            src_dev = lax.rem(my - k + N_DEV, N_DEV)
            recv = pltpu.make_async_remote_copy(
                src_ref=msbuf.at[src_dev],
                dst_ref=msbuf.at[src_dev],
                send_sem=ms_send.at[k - 1],
                recv_sem=ms_recv.at[src_dev],
                device_id=(my,),
                device_id_type=pl.DeviceIdType.MESH,
            )
            recv.wait_recv()
        for r in sends:
            r.wait_send()

        M = msbuf[0, :, 0:1]
        for c in range(1, N_DEV):
            M = jnp.maximum(M, msbuf[c, :, 0:1])
        S = jnp.zeros((t, 1), dtype=jnp.float32)
        for c in range(N_DEV):
            S = S + msbuf[c, :, 1:2] * jnp.exp(msbuf[c, :, 0:1] - M)
        cbuf[0] = cbuf[0] * (jnp.exp(m_loc - M) * (1.0 / S))
        cp = pltpu.make_async_copy(
            cbuf.at[0], out_ref.at[:, pl.ds(my * v_per, v_per)], csem
        )
        cp.start()

        v_half = v_per // 2
        v_q = v_per // 4

        def mk_hop(h, direction, q):
            if direction == 0:
                origin = lax.rem(my - h + 2 * N_DEV, N_DEV)
                col = q * v_q
                tgt = right
                ssem, rsem = ring_send, ring_recv
            else:
                origin = lax.rem(my + h, N_DEV)
                col = v_half + q * v_q
                tgt = left
                ssem, rsem = ring2_send, ring2_recv
            sl = pl.ds(origin * v_per + col, v_q)
            if h == 0:
                src = cbuf.at[0, :, pl.ds(col, v_q)]
            else:
                src = out_ref.at[:, sl]
            return pltpu.make_async_remote_copy(
                src_ref=src,
                dst_ref=out_ref.at[:, sl],
                send_sem=ssem.at[h, q],
                recv_sem=rsem.at[h, q],
                device_id=(tgt,),
                device_id_type=pl.DeviceIdType.MESH,
            )

        streams = {}
        for q in range(2):
            for direction in range(2):
                r = mk_hop(0, direction, q)
                r.start()
                streams[(direction, q)] = [r]
        for h in range(N_DEV - 1):
            for q in range(2):
                for direction in range(2):
                    streams[(direction, q)][h].wait_recv()
                    if h < N_DEV - 2:
                        r = mk_hop(h + 1, direction, q)
                        r.start()
                        streams[(direction, q)].append(r)
        for lst in streams.values():
            for r in lst:
                r.wait_send()
        cp.wait()

    return pl.pallas_call(
        body,
        out_shape=jax.ShapeDtypeStruct((t, v_total), jnp.float32),
        in_specs=[
            pl.BlockSpec(memory_space=pltpu.VMEM),
            pl.BlockSpec(memory_space=pltpu.VMEM),
        ],
        out_specs=pl.BlockSpec(memory_space=pltpu.MemorySpace.HBM),
        scratch_shapes=[
            pltpu.VMEM((1, t, v_per), jnp.float32),
            pltpu.VMEM((N_DEV, t, 128), jnp.float32),
            pltpu.SemaphoreType.DMA((N_DEV - 1,)),
            pltpu.SemaphoreType.DMA((N_DEV,)),
            pltpu.SemaphoreType.DMA((N_DEV - 1, 2)),
            pltpu.SemaphoreType.DMA((N_DEV - 1, 2)),
            pltpu.SemaphoreType.DMA((N_DEV - 1, 2)),
            pltpu.SemaphoreType.DMA((N_DEV - 1, 2)),
            pltpu.SemaphoreType.DMA,
        ],
        compiler_params=pltpu.CompilerParams(collective_id=0),
    )(x, W)
